# Initial kernel scaffold; baseline (speedup 1.0000x reference)
import jax
import jax.numpy as jnp
from jax.experimental import pallas as pl


def kernel(X, emb_tables, W1, b1, W2, b2, W3, b3):
    raise NotImplementedError("write your pallas kernel here")


# speedup vs baseline: 8.0133x; 8.0133x over previous
import functools

import jax
import jax.numpy as jnp
from jax import lax
from jax.experimental import pallas as pl
from jax.experimental.pallas import tpu as pltpu
from jax.experimental.pallas import tpu_sc as plsc

N_CONT = 13
N_CAT = 26
VOCAB = 100000
EMB_DIM = 32
BATCH = 16384
ROWS = BATCH * N_CAT

NW = 32
R_PER_W = ROWS // NW
CHUNK = 1024
N_CHUNK = R_PER_W // CHUNK
SUB = 128
N_SUB = CHUNK // SUB


def _gather_body(idx_hbm, tab_hbm, out_hbm, idx_v, rows_v, sem):
    wid = lax.axis_index("s") * 2 + lax.axis_index("c")
    base_w = wid * R_PER_W

    def chunk_body(i, carry):
        base = pl.multiple_of(base_w + i * CHUNK, CHUNK)
        pltpu.sync_copy(idx_hbm.at[pl.ds(base, CHUNK)], idx_v)
        copies = [
            pltpu.async_copy(
                tab_hbm.at[idx_v.at[pl.ds(j * SUB, SUB)]],
                rows_v.at[pl.ds(j * SUB, SUB)],
                sem,
            )
            for j in range(N_SUB)
        ]
        for c in copies:
            c.wait()
        pltpu.sync_copy(rows_v, out_hbm.at[pl.ds(base, CHUNK)])
        return carry

    lax.fori_loop(0, N_CHUNK, chunk_body, 0)


@functools.cache
def _sc_gather():
    return pl.kernel(
        _gather_body,
        out_type=jax.ShapeDtypeStruct((ROWS, EMB_DIM), jnp.float32),
        mesh=plsc.VectorSubcoreMesh(core_axis_name="c", subcore_axis_name="s"),
        scratch_types=[
            pltpu.VMEM((CHUNK,), jnp.int32),
            pltpu.VMEM((CHUNK, EMB_DIM), jnp.float32),
            pltpu.SemaphoreType.DMA,
        ],
        compiler_params=pltpu.CompilerParams(use_tc_tiling_on_sc=False),
    )


BLK = 1024


def _mlp_body(x_ref, emb_ref, w1c_ref, w1e_ref, b1_ref, w2_ref, b2_ref,
              w3_ref, b3_ref, out_ref):
    cont = x_ref[:, :N_CONT].astype(jnp.float32)
    h = jnp.dot(cont, w1c_ref[:], preferred_element_type=jnp.float32)
    h += jnp.dot(emb_ref[:], w1e_ref[:], preferred_element_type=jnp.float32)
    h = jnp.maximum(h + b1_ref[:], 0.0)
    h = jnp.maximum(
        jnp.dot(h, w2_ref[:], preferred_element_type=jnp.float32) + b2_ref[:], 0.0)
    out_ref[:] = jnp.dot(h, w3_ref[:], preferred_element_type=jnp.float32) + b3_ref[:]


def _mlp(x, emb, w1c, w1e, b1, w2, b2, w3, b3):
    grid = (BATCH // BLK,)
    full = lambda shape: pl.BlockSpec(shape, lambda i: (0, 0))
    return pl.pallas_call(
        _mlp_body,
        grid=grid,
        in_specs=[
            pl.BlockSpec((BLK, N_CONT + N_CAT), lambda i: (i, 0)),
            pl.BlockSpec((BLK, N_CAT * EMB_DIM), lambda i: (i, 0)),
            full(w1c.shape),
            full(w1e.shape),
            full(b1.shape),
            full(w2.shape),
            full(b2.shape),
            full(w3.shape),
            full(b3.shape),
        ],
        out_specs=pl.BlockSpec((BLK, 1), lambda i: (i, 0)),
        out_shape=jax.ShapeDtypeStruct((BATCH, 1), jnp.float32),
    )(x, emb, w1c, w1e, b1, w2, b2, w3, b3)


def kernel(X, emb_tables, W1, b1, W2, b2, W3, b3):
    x_cat = X[:, N_CONT:].astype(jnp.int32)
    offs = (jnp.arange(N_CAT, dtype=jnp.int32) * VOCAB)[None, :]
    flat_idx = (x_cat + offs).reshape(-1)
    tab_flat = emb_tables.reshape(N_CAT * VOCAB, EMB_DIM)

    emb = _sc_gather()(flat_idx, tab_flat)

    out = _mlp(
        X, emb.reshape(BATCH, N_CAT * EMB_DIM),
        W1[:N_CONT], W1[N_CONT:], b1.reshape(1, -1),
        W2, b2.reshape(1, -1), W3, b3.reshape(1, 1),
    )
    return out.reshape(BATCH)


# speedup vs baseline: 18.2196x; 2.2737x over previous
import functools

import jax
import jax.numpy as jnp
from jax import lax
from jax.experimental import pallas as pl
from jax.experimental.pallas import tpu as pltpu
from jax.experimental.pallas import tpu_sc as plsc

N_CONT = 13
N_CAT = 26
VOCAB = 100000
EMB_DIM = 32
BATCH = 16384
ROWS = BATCH * N_CAT
H1 = 64
NPAIR = N_CAT // 2

VB = 2048
NV = -(-VOCAB // VB)
VOCAB_PAD = NV * VB
G2_ROWS = NPAIR * VOCAB_PAD


def _g2_body(tab_ref, w_ref, out_ref):
    for j in range(NPAIR):
        out_ref[j] = lax.dot_general(
            tab_ref[pl.ds(j * 2 * EMB_DIM, 2 * EMB_DIM)], w_ref[pl.ds(j * 2 * EMB_DIM, 2 * EMB_DIM)],
            (((0,), (0,)), ((), ())),
            preferred_element_type=jnp.float32)


def _g2(tabT, w1p):
    return pl.pallas_call(
        _g2_body,
        grid=(NV,),
        in_specs=[
            pl.BlockSpec((N_CAT * EMB_DIM, VB), lambda v: (0, v)),
            pl.BlockSpec((N_CAT * EMB_DIM, 128), lambda v: (0, 0)),
        ],
        out_specs=pl.BlockSpec((NPAIR, VB, 128), lambda v: (0, v, 0)),
        out_shape=jax.ShapeDtypeStruct((NPAIR, VOCAB_PAD, 128), jnp.float32),
        compiler_params=pltpu.CompilerParams(
            dimension_semantics=("arbitrary",)),
    )(tabT, w1p)


HBATCH = BATCH // 2
HROWS = N_CAT * HBATCH
NW = 32
R_PER_W = HROWS // NW
CHUNK = 512
N_CHUNK = R_PER_W // CHUNK
SUB = 128
N_SUB = CHUNK // SUB


def _gather_body(idx_hbm, g2_hbm, out_hbm, idx_v, rows_v, sem):
    wid = lax.axis_index("s") * 2 + lax.axis_index("c")
    base_w = wid * R_PER_W

    def chunk_body(i, carry):
        base = pl.multiple_of(base_w + i * CHUNK, CHUNK)
        pltpu.sync_copy(idx_hbm.at[pl.ds(base, CHUNK)], idx_v)
        copies = [
            pltpu.async_copy(
                g2_hbm.at[idx_v.at[pl.ds(j * SUB, SUB)]],
                rows_v.at[pl.ds(j * SUB, SUB)],
                sem,
            )
            for j in range(N_SUB)
        ]
        for c in copies:
            c.wait()
        pltpu.sync_copy(rows_v, out_hbm.at[pl.ds(base, CHUNK)])
        return carry

    lax.fori_loop(0, N_CHUNK, chunk_body, 0)


@functools.cache
def _sc_gather():
    return pl.kernel(
        _gather_body,
        out_type=jax.ShapeDtypeStruct((HROWS, 128), jnp.float32),
        mesh=plsc.VectorSubcoreMesh(core_axis_name="c", subcore_axis_name="s"),
        scratch_types=[
            pltpu.VMEM((CHUNK,), jnp.int32),
            pltpu.VMEM((CHUNK, 128), jnp.float32),
            pltpu.SemaphoreType.DMA,
        ],
    )


BLK = 512


def _mlp_body(xt_ref, emb_ref, w1c_ref, b1_ref, w2_ref, b2_ref,
              w3_ref, b3_ref, out_ref):
    he = emb_ref[0, :, :H1]
    for f in range(1, N_CAT):
        he = he + (emb_ref[f, :, H1:] if f % 2 else emb_ref[f, :, :H1])
    contT = xt_ref[:][:N_CONT].astype(jnp.float32)
    h = lax.dot_general(contT, w1c_ref[:], (((0,), (0,)), ((), ())),
                        preferred_element_type=jnp.float32)
    h = jnp.maximum(h + he + b1_ref[:], 0.0)
    h = jnp.maximum(
        jnp.dot(h, w2_ref[:], preferred_element_type=jnp.float32) + b2_ref[:], 0.0)
    out_ref[:] = jnp.dot(h, w3_ref[:], preferred_element_type=jnp.float32) + b3_ref[:]


def _mlp(xt, emb3, half, w1c, b1, w2, b2, w3, b3):
    full = lambda shape: pl.BlockSpec(shape, lambda i: (0, 0))
    koff = half * (HBATCH // BLK)
    return pl.pallas_call(
        _mlp_body,
        grid=(HBATCH // BLK,),
        in_specs=[
            pl.BlockSpec((N_CONT + N_CAT, BLK), lambda i: (0, koff + i)),
            pl.BlockSpec((N_CAT, BLK, 128), lambda i: (0, i, 0)),
            full(w1c.shape),
            full(b1.shape),
            full(w2.shape),
            full(b2.shape),
            full(w3.shape),
            full(b3.shape),
        ],
        out_specs=pl.BlockSpec((BLK, 1), lambda i: (i, 0)),
        out_shape=jax.ShapeDtypeStruct((HBATCH, 1), jnp.float32),
    )(xt, emb3, w1c, b1, w2, b2, w3, b3)


def kernel(X, emb_tables, W1, b1, W2, b2, W3, b3):
    tabT = emb_tables.transpose(0, 2, 1).reshape(N_CAT * EMB_DIM, VOCAB)
    XT = X.T

    w1e = W1[N_CONT:].reshape(N_CAT, EMB_DIM, H1)
    zeros = jnp.zeros_like(w1e)
    w1p = jnp.where(
        (jnp.arange(N_CAT) % 2 == 0)[:, None, None],
        jnp.concatenate([w1e, zeros], axis=-1),
        jnp.concatenate([zeros, w1e], axis=-1),
    ).reshape(N_CAT * EMB_DIM, 2 * H1)

    g2 = _g2(tabT, w1p).reshape(G2_ROWS, 128)

    x_catT = XT[N_CONT:].astype(jnp.int32)
    base = ((jnp.arange(N_CAT, dtype=jnp.int32) // 2) * VOCAB_PAD)[:, None]
    outs = []
    for half in range(2):
        idx = (x_catT[:, half * HBATCH:(half + 1) * HBATCH] + base).reshape(-1)
        emb = _sc_gather()(idx, g2)
        outs.append(_mlp(
            XT, emb.reshape(N_CAT, HBATCH, 128), half,
            W1[:N_CONT], b1.reshape(1, -1),
            W2, b2.reshape(1, -1), W3, b3.reshape(1, 1),
        ))
    return jnp.concatenate(outs, axis=0).reshape(BATCH)
